# baseline (speedup 1.0000x reference)
import functools

import jax
import jax.numpy as jnp
import numpy as np
from jax import lax
from jax.experimental import pallas as pl
from jax.experimental.pallas import tpu as pltpu
from jax.experimental.pallas import tpu_sc as plsc

_NUM_WORKERS = 32
_LANES = 16


def _position_encoding(max_len: int, d_model: int) -> np.ndarray:
    angle_rates = 10000.0 ** (2.0 * (np.arange(d_model, dtype=np.float64) / d_model))
    angle = np.arange(max_len, dtype=np.float64)[:, None] / angle_rates
    values = np.stack([np.sin(angle[:, 0::2]), np.cos(angle[:, 1::2])], axis=2)
    return values.reshape(max_len, -1).astype(np.float32)


@functools.lru_cache(maxsize=None)
def _build(batch: int, seq_len: int, voc: int, d_model: int):
    assert batch == 1024 and seq_len % 8 == 0 and d_model == 64
    n_bh = batch // 128
    n_units = seq_len * n_bh
    assert n_units % _NUM_WORKERS == 0
    units_per_w = n_units // _NUM_WORKERS
    scale = float(np.sqrt(d_model))

    mesh = plsc.VectorSubcoreMesh(
        core_axis_name="c", subcore_axis_name="s", num_cores=2, num_subcores=16)

    def body(seq5, table, pe_hbm, out5,
             idx0, idx1, rows0, rows1, tile0, tile1, pe_v,
             gsem0, gsem1, ssem0, ssem1):
        w = lax.axis_index("s") * 2 + lax.axis_index("c")
        base_u = w * units_per_w

        pltpu.sync_copy(pe_hbm, pe_v)

        lane = lax.iota(jnp.int32, _LANES)
        dl_vec = lane & 7
        lane_hi = lane >> 3
        dh_vecs = [lane_hi + (2 * s) for s in range(4)]

        bufs = ((idx0, rows0, tile0, gsem0, ssem0),
                (idx1, rows1, tile1, gsem1, ssem1))

        def load_idx(g, idx_v):
            l = g >> 3
            bh = g & 7
            pltpu.sync_copy(seq5.at[l >> 3, bh, l & 7], idx_v)

        def unit_body(u, cur, nxt):
            idx_c, rows_c, tile_c, gsem_c, ssem_c = cur
            idx_n, rows_n, tile_n, gsem_n, ssem_n = nxt
            g = base_u + u
            l = g >> 3
            bh = g & 7

            @pl.when(u < units_per_w - 1)
            def _():
                load_idx(g + 1, idx_n)
                pltpu.async_copy(table.at[idx_n], rows_n, gsem_n)

            pltpu.make_async_copy(table.at[idx_c], rows_c, gsem_c).wait()

            @pl.when(u >= 2)
            def _():
                pltpu.make_async_copy(tile_c, out5.at[l, :, bh], ssem_c).wait()

            pe_s = [pe_v[l, pl.ds(s * _LANES, _LANES)] for s in range(4)]

            def rbody(r, carry):
                r_vec = jnp.broadcast_to(r, (_LANES,))
                for s in range(4):
                    val = rows_c[r, pl.ds(s * _LANES, _LANES)] * scale + pe_s[s]
                    plsc.store_scatter(tile_c, [dh_vecs[s], dl_vec, r_vec], val)
                return carry
            lax.fori_loop(0, 128, rbody, 0)

            pltpu.async_copy(tile_c, out5.at[l, :, bh], ssem_c)

        load_idx(base_u, idx0)
        pltpu.async_copy(table.at[idx0], rows0, gsem0)

        def loop_body(u, carry):
            @pl.when((u & 1) == 0)
            def _():
                unit_body(u, bufs[0], bufs[1])

            @pl.when((u & 1) == 1)
            def _():
                unit_body(u, bufs[1], bufs[0])
            return carry
        lax.fori_loop(0, units_per_w, loop_body, 0)

        for last in (units_per_w - 2, units_per_w - 1):
            g = base_u + last
            l = g >> 3
            bh = g & 7
            tile_l, ssem_l = bufs[last % 2][2], bufs[last % 2][4]
            pltpu.make_async_copy(tile_l, out5.at[l, :, bh], ssem_l).wait()

    run = pl.kernel(
        body,
        out_type=jax.ShapeDtypeStruct((seq_len, 8, n_bh, 8, 128), jnp.float32),
        mesh=mesh,
        compiler_params=pltpu.CompilerParams(
            use_tc_tiling_on_sc=False, needs_layout_passes=False),
        scratch_types=[
            pltpu.VMEM((128,), jnp.int32),
            pltpu.VMEM((128,), jnp.int32),
            pltpu.VMEM((128, d_model), jnp.float32),
            pltpu.VMEM((128, d_model), jnp.float32),
            pltpu.VMEM((8, 8, 128), jnp.float32),
            pltpu.VMEM((8, 8, 128), jnp.float32),
            pltpu.VMEM((seq_len, d_model), jnp.float32),
            pltpu.SemaphoreType.DMA,
            pltpu.SemaphoreType.DMA,
            pltpu.SemaphoreType.DMA,
            pltpu.SemaphoreType.DMA,
        ],
    )
    return run


def kernel(sequences, emb_table):
    batch, seq_len = sequences.shape
    voc, d_model = emb_table.shape
    pe = jnp.asarray(_position_encoding(seq_len, d_model))
    run = _build(batch, seq_len, voc, d_model)
    seq5 = sequences.reshape(8, 128, seq_len // 8, 8).transpose(2, 0, 3, 1)
    out5 = run(seq5, emb_table, pe)
    out = out5.transpose(2, 4, 0, 1, 3).reshape(batch, seq_len, d_model)
    return out


# speedup vs baseline: 2.4113x; 2.4113x over previous
import functools

import jax
import jax.numpy as jnp
import numpy as np
from jax import lax
from jax.experimental import pallas as pl
from jax.experimental.pallas import tpu as pltpu
from jax.experimental.pallas import tpu_sc as plsc

_NUM_WORKERS = 32
_LANES = 16


def _position_encoding(max_len: int, d_model: int) -> np.ndarray:
    angle_rates = 10000.0 ** (2.0 * (np.arange(d_model, dtype=np.float64) / d_model))
    angle = np.arange(max_len, dtype=np.float64)[:, None] / angle_rates
    values = np.stack([np.sin(angle[:, 0::2]), np.cos(angle[:, 1::2])], axis=2)
    return values.reshape(max_len, -1).astype(np.float32)


@functools.lru_cache(maxsize=None)
def _build(batch: int, seq_len: int, voc: int, d_model: int):
    assert batch == 1024 and seq_len % 8 == 0 and d_model == 64
    n_bh = batch // 128
    n_units = seq_len * n_bh
    assert n_units % _NUM_WORKERS == 0
    units_per_w = n_units // _NUM_WORKERS
    scale = float(np.sqrt(d_model))

    mesh = plsc.VectorSubcoreMesh(
        core_axis_name="c", subcore_axis_name="s", num_cores=2, num_subcores=16)

    def body(seq5, table, pe_hbm, out5,
             idx0, idx1, rows0, rows1, stg0, stg1, tile0, tile1, pe_v,
             gsem0, gsem1, ssem0, ssem1):
        w = lax.axis_index("s") * 2 + lax.axis_index("c")
        base_u = w * units_per_w

        pltpu.sync_copy(pe_hbm, pe_v)

        lane = lax.iota(jnp.int32, _LANES)
        row_vecs = [lane + b0 for b0 in range(0, 128, _LANES)]

        bufs = ((idx0, rows0, stg0, tile0, gsem0, ssem0),
                (idx1, rows1, stg1, tile1, gsem1, ssem1))

        def load_idx(g, idx_v):
            l = g >> 3
            bh = g & 7
            pltpu.sync_copy(seq5.at[l >> 3, bh, l & 7], idx_v)

        def unit_body(u, cur, nxt):
            idx_c, rows_c, stg_c, tile_c, gsem_c, ssem_c = cur
            idx_n, rows_n, stg_n, tile_n, gsem_n, ssem_n = nxt
            g = base_u + u
            l = g >> 3
            bh = g & 7

            @pl.when(u < units_per_w - 1)
            def _():
                load_idx(g + 1, idx_n)
                pltpu.async_copy(table.at[idx_n], rows_n, gsem_n)

            pltpu.make_async_copy(table.at[idx_c], rows_c, gsem_c).wait()

            @pl.when(u >= 2)
            def _():
                pltpu.make_async_copy(tile_c, out5.at[l, :, bh], ssem_c).wait()

            pe_s = [pe_v[l, pl.ds(s * _LANES, _LANES)] for s in range(4)]

            @plsc.parallel_loop(0, 128, 1, unroll=8)
            def _(r):
                for s in range(4):
                    sl = pl.ds(s * _LANES, _LANES)
                    stg_c[r, sl] = rows_c[r, sl] * scale + pe_s[s]

            @plsc.parallel_loop(0, 64, 1, unroll=4)
            def _(d):
                d_vec = jnp.broadcast_to(d, (_LANES,))
                dh = d >> 3
                dl = d & 7
                for b0 in range(8):
                    val = plsc.load_gather(stg_c, [row_vecs[b0], d_vec])
                    tile_c[dh, dl, pl.ds(b0 * _LANES, _LANES)] = val

            pltpu.async_copy(tile_c, out5.at[l, :, bh], ssem_c)

        load_idx(base_u, idx0)
        pltpu.async_copy(table.at[idx0], rows0, gsem0)

        def loop_body(u, carry):
            @pl.when((u & 1) == 0)
            def _():
                unit_body(u, bufs[0], bufs[1])

            @pl.when((u & 1) == 1)
            def _():
                unit_body(u, bufs[1], bufs[0])
            return carry
        lax.fori_loop(0, units_per_w, loop_body, 0)

        for last in (units_per_w - 2, units_per_w - 1):
            g = base_u + last
            l = g >> 3
            bh = g & 7
            tile_l, ssem_l = bufs[last % 2][3], bufs[last % 2][5]
            pltpu.make_async_copy(tile_l, out5.at[l, :, bh], ssem_l).wait()

    run = pl.kernel(
        body,
        out_type=jax.ShapeDtypeStruct((seq_len, 8, n_bh, 8, 128), jnp.float32),
        mesh=mesh,
        compiler_params=pltpu.CompilerParams(
            use_tc_tiling_on_sc=False, needs_layout_passes=False,
            disable_bounds_checks=True),
        scratch_types=[
            pltpu.VMEM((128,), jnp.int32),
            pltpu.VMEM((128,), jnp.int32),
            pltpu.VMEM((128, d_model), jnp.float32),
            pltpu.VMEM((128, d_model), jnp.float32),
            pltpu.VMEM((128, 80), jnp.float32),
            pltpu.VMEM((128, 80), jnp.float32),
            pltpu.VMEM((8, 8, 128), jnp.float32),
            pltpu.VMEM((8, 8, 128), jnp.float32),
            pltpu.VMEM((seq_len, d_model), jnp.float32),
            pltpu.SemaphoreType.DMA,
            pltpu.SemaphoreType.DMA,
            pltpu.SemaphoreType.DMA,
            pltpu.SemaphoreType.DMA,
        ],
    )
    return run


def kernel(sequences, emb_table):
    batch, seq_len = sequences.shape
    voc, d_model = emb_table.shape
    pe = jnp.asarray(_position_encoding(seq_len, d_model))
    run = _build(batch, seq_len, voc, d_model)
    seq5 = sequences.reshape(8, 128, seq_len // 8, 8).transpose(2, 0, 3, 1)
    out5 = run(seq5, emb_table, pe)
    out = out5.transpose(2, 4, 0, 1, 3).reshape(batch, seq_len, d_model)
    return out
